# baseline (device time: 43164 ns/iter reference)
import jax
import jax.numpy as jnp
from jax import lax
from jax.experimental import pallas as pl
from jax.experimental.pallas import tpu as pltpu

N_DEV = 4
N_RDMA = 18


def kernel(x, Win0, Wout0, Win1, Wout1, Win2, Wout2):
    m_per, d = x.shape
    _, h_per = Win0.shape

    def body(x_ref, win0, wout0, win1, wout1, win2, wout2, out_ref,
             XS, WS0, WS1, WS2, VS0, VS1, VS2,
             W0, W1, W2, V0, V1, V2,
             local_sems, send_sems, recv_sems):
        me = lax.axis_index("i")
        ypart = me ^ 1
        xpart = 3 - me

        sem = iter(range(N_RDMA))

        def start(*quads):
            rdmas = []
            for src, dst, b, partner in quads:
                i = next(sem)
                r = pltpu.make_async_remote_copy(
                    src_ref=src.at[b],
                    dst_ref=dst.at[b],
                    send_sem=send_sems.at[i],
                    recv_sem=recv_sems.at[i],
                    device_id=(partner,),
                    device_id_type=pl.DeviceIdType.MESH,
                )
                r.start()
                rdmas.append(r)
            return rdmas

        def wait(rdmas):
            for r in rdmas:
                r.wait()

        fetches = []
        for i, (src, dst) in enumerate((
            (win0, WS0), (wout0, VS0), (win1, WS1), (wout1, VS1),
            (win2, WS2), (wout2, VS2), (x_ref, XS),
        )):
            c = pltpu.make_async_copy(src, dst, local_sems.at[i])
            c.start()
            fetches.append(c)

        barrier = pltpu.get_barrier_semaphore()
        for nbr in (ypart, xpart):
            pl.semaphore_signal(
                barrier, inc=1,
                device_id=(nbr,), device_id_type=pl.DeviceIdType.MESH,
            )
        pl.semaphore_wait(barrier, 2)

        r1 = []
        for l, (W, V, WS, VS) in enumerate((
            (W0, V0, WS0, VS0), (W1, V1, WS1, VS1), (W2, V2, WS2, VS2),
        )):
            fetches[2 * l].wait()
            W[pl.ds(me, 1)] = WS[...].astype(jnp.bfloat16)[None]
            fetches[2 * l + 1].wait()
            V[pl.ds(me, 1)] = VS[...].astype(jnp.bfloat16)[None]
            r1.append(start((W, W, me, ypart), (V, V, me, xpart)))

        r2 = []
        for (W, V), r in zip(((W0, V0), (W1, V1), (W2, V2)), r1):
            wait(r)
            r2.append(start(
                (W, W, me, xpart), (W, W, ypart, xpart),
                (V, V, me, ypart), (V, V, xpart, ypart),
            ))

        fetches[6].wait()
        xl = XS[...].astype(jnp.bfloat16)
        for l, (W, V) in enumerate(((W0, V0), (W1, V1), (W2, V2))):
            wait(r2[l])
            h3 = lax.dot_general(
                xl, W[...],
                dimension_numbers=(((1,), (1,)), ((), ())),
                preferred_element_type=jnp.float32,
            )
            hb = (
                jnp.maximum(h3, 0.0)
                .astype(jnp.bfloat16)
                .reshape(m_per, N_DEV * h_per)
            )
            acc = jnp.dot(
                hb, V[...].reshape(N_DEV * h_per, d),
                preferred_element_type=jnp.float32,
            )
            if l < 2:
                xl = acc.astype(jnp.bfloat16)
            else:
                out_ref[...] = acc

    return pl.pallas_call(
        body,
        out_shape=jax.ShapeDtypeStruct((m_per, d), jnp.float32),
        in_specs=[pl.BlockSpec(memory_space=pl.MemorySpace.ANY)] * 7,
        out_specs=pl.BlockSpec(memory_space=pltpu.VMEM),
        scratch_shapes=[
            pltpu.VMEM((m_per, d), jnp.float32),
            pltpu.VMEM((d, h_per), jnp.float32),
            pltpu.VMEM((d, h_per), jnp.float32),
            pltpu.VMEM((d, h_per), jnp.float32),
            pltpu.VMEM((h_per, d), jnp.float32),
            pltpu.VMEM((h_per, d), jnp.float32),
            pltpu.VMEM((h_per, d), jnp.float32),
            pltpu.VMEM((N_DEV, d, h_per), jnp.bfloat16),
            pltpu.VMEM((N_DEV, d, h_per), jnp.bfloat16),
            pltpu.VMEM((N_DEV, d, h_per), jnp.bfloat16),
            pltpu.VMEM((N_DEV, h_per, d), jnp.bfloat16),
            pltpu.VMEM((N_DEV, h_per, d), jnp.bfloat16),
            pltpu.VMEM((N_DEV, h_per, d), jnp.bfloat16),
            pltpu.SemaphoreType.DMA((7,)),
            pltpu.SemaphoreType.DMA((N_RDMA,)),
            pltpu.SemaphoreType.DMA((N_RDMA,)),
        ],
        compiler_params=pltpu.CompilerParams(collective_id=0),
    )(x, Win0, Wout0, Win1, Wout1, Win2, Wout2)


# device time: 40725 ns/iter; 1.0599x vs baseline; 1.0599x over previous
import jax
import jax.numpy as jnp
from jax import lax
from jax.experimental import pallas as pl
from jax.experimental.pallas import tpu as pltpu

N_DEV = 4
N_RDMA = 18


def kernel(x, Win0, Wout0, Win1, Wout1, Win2, Wout2):
    m_per, d = x.shape
    _, h_per = Win0.shape

    def body(x_ref, win0, wout0, win1, wout1, win2, wout2, out_ref,
             W0, W1, W2, V0, V1, V2, send_sems, recv_sems):
        me = lax.axis_index("i")
        ypart = me ^ 1
        xpart = 3 - me

        sem = iter(range(N_RDMA))

        def start(*quads):
            rdmas = []
            for src, dst, b, partner in quads:
                i = next(sem)
                r = pltpu.make_async_remote_copy(
                    src_ref=src.at[b],
                    dst_ref=dst.at[b],
                    send_sem=send_sems.at[i],
                    recv_sem=recv_sems.at[i],
                    device_id=(partner,),
                    device_id_type=pl.DeviceIdType.MESH,
                )
                r.start()
                rdmas.append(r)
            return rdmas

        def wait(rdmas):
            for r in rdmas:
                r.wait()

        for ref, src_ref in (
            (W0, win0), (W1, win1), (W2, win2),
            (V0, wout0), (V1, wout1), (V2, wout2),
        ):
            ref[pl.ds(me, 1)] = src_ref[...].astype(jnp.bfloat16)[None]
        xl = x_ref[...].astype(jnp.bfloat16)

        barrier = pltpu.get_barrier_semaphore()
        for nbr in (ypart, xpart):
            pl.semaphore_signal(
                barrier, inc=1,
                device_id=(nbr,), device_id_type=pl.DeviceIdType.MESH,
            )
        pl.semaphore_wait(barrier, 2)

        r1 = []
        for W, V in ((W0, V0), (W1, V1), (W2, V2)):
            r1.append(start(
                (W, W, me, ypart), (V, V, me, xpart),
                (W, W, me, xpart), (V, V, me, ypart),
            ))
        r2 = []
        for (W, V), r in zip(((W0, V0), (W1, V1), (W2, V2)), r1):
            wait(r)
            r2.append(start(
                (W, W, ypart, xpart), (V, V, xpart, ypart),
            ))

        for l, (W, V) in enumerate(((W0, V0), (W1, V1), (W2, V2))):
            wait(r2[l])
            acc = jnp.zeros((m_per, d), jnp.float32)
            for q in range(N_DEV):
                hq = jnp.maximum(
                    jnp.dot(xl, W[q], preferred_element_type=jnp.float32),
                    0.0,
                ).astype(jnp.bfloat16)
                acc = acc + jnp.dot(
                    hq, V[q], preferred_element_type=jnp.float32
                )
            if l < 2:
                xl = acc.astype(jnp.bfloat16)
            else:
                out_ref[...] = acc

    return pl.pallas_call(
        body,
        out_shape=jax.ShapeDtypeStruct((m_per, d), jnp.float32),
        in_specs=[pl.BlockSpec(memory_space=pltpu.VMEM)] * 7,
        out_specs=pl.BlockSpec(memory_space=pltpu.VMEM),
        scratch_shapes=[
            pltpu.VMEM((N_DEV, d, h_per), jnp.bfloat16),
            pltpu.VMEM((N_DEV, d, h_per), jnp.bfloat16),
            pltpu.VMEM((N_DEV, d, h_per), jnp.bfloat16),
            pltpu.VMEM((N_DEV, h_per, d), jnp.bfloat16),
            pltpu.VMEM((N_DEV, h_per, d), jnp.bfloat16),
            pltpu.VMEM((N_DEV, h_per, d), jnp.bfloat16),
            pltpu.SemaphoreType.DMA((N_RDMA,)),
            pltpu.SemaphoreType.DMA((N_RDMA,)),
        ],
        compiler_params=pltpu.CompilerParams(collective_id=0),
    )(x, Win0, Wout0, Win1, Wout1, Win2, Wout2)


# device time: 39454 ns/iter; 1.0940x vs baseline; 1.0322x over previous
import jax
import jax.numpy as jnp
from jax import lax
from jax.experimental import pallas as pl
from jax.experimental.pallas import tpu as pltpu

N_DEV = 4
N_RDMA = 18


def kernel(x, Win0, Wout0, Win1, Wout1, Win2, Wout2):
    m_per, d = x.shape
    _, h_per = Win0.shape

    def body(x_ref, win0, wout0, win1, wout1, win2, wout2, out_ref,
             W0, W1, W2, V0, V1, V2, send_sems, recv_sems):
        me = lax.axis_index("i")
        ypart = me ^ 1
        xpart = 3 - me

        sem = iter(range(N_RDMA))

        def start(*quads):
            rdmas = []
            for src, dst, b, partner in quads:
                i = next(sem)
                r = pltpu.make_async_remote_copy(
                    src_ref=src.at[b],
                    dst_ref=dst.at[b],
                    send_sem=send_sems.at[i],
                    recv_sem=recv_sems.at[i],
                    device_id=(partner,),
                    device_id_type=pl.DeviceIdType.MESH,
                )
                r.start()
                rdmas.append(r)
            return rdmas

        def wait(rdmas):
            for r in rdmas:
                r.wait()

        for ref, src_ref in (
            (W0, win0), (W1, win1), (W2, win2),
            (V0, wout0), (V1, wout1), (V2, wout2),
        ):
            ref[pl.ds(me, 1)] = src_ref[...].astype(jnp.bfloat16)[None]
        xl = x_ref[...].astype(jnp.bfloat16)

        barrier = pltpu.get_barrier_semaphore()
        for nbr in (ypart, xpart):
            pl.semaphore_signal(
                barrier, inc=1,
                device_id=(nbr,), device_id_type=pl.DeviceIdType.MESH,
            )
        pl.semaphore_wait(barrier, 2)

        def own(W, V):
            return start(
                (W, W, me, ypart), (V, V, me, xpart),
                (W, W, me, xpart), (V, V, me, ypart),
            )

        def fwd(W, V):
            return start(
                (W, W, ypart, xpart), (V, V, xpart, ypart),
            )

        def chunk(W, V, q, xv):
            hq = jnp.maximum(
                jnp.dot(
                    xv, W[pl.ds(q, 1)][0],
                    preferred_element_type=jnp.float32,
                ),
                0.0,
            ).astype(jnp.bfloat16)
            return jnp.dot(
                hq, V[pl.ds(q, 1)][0], preferred_element_type=jnp.float32
            )

        def early(W, V, xv):
            return (
                chunk(W, V, me, xv)
                + chunk(W, V, ypart, xv)
                + chunk(W, V, xpart, xv)
            )

        r1_0 = own(W0, V0)
        r1_1 = own(W1, V1)
        wait(r1_0)
        f0 = fwd(W0, V0)
        r1_2 = own(W2, V2)
        acc0 = early(W0, V0, xl)
        wait(r1_1)
        f1 = fwd(W1, V1)
        wait(f0)
        xl1 = (acc0 + chunk(W0, V0, me ^ 2, xl)).astype(jnp.bfloat16)
        acc1 = early(W1, V1, xl1)
        wait(r1_2)
        f2 = fwd(W2, V2)
        wait(f1)
        xl2 = (acc1 + chunk(W1, V1, me ^ 2, xl1)).astype(jnp.bfloat16)
        acc2 = early(W2, V2, xl2)
        wait(f2)
        out_ref[...] = acc2 + chunk(W2, V2, me ^ 2, xl2)

    return pl.pallas_call(
        body,
        out_shape=jax.ShapeDtypeStruct((m_per, d), jnp.float32),
        in_specs=[pl.BlockSpec(memory_space=pltpu.VMEM)] * 7,
        out_specs=pl.BlockSpec(memory_space=pltpu.VMEM),
        scratch_shapes=[
            pltpu.VMEM((N_DEV, d, h_per), jnp.bfloat16),
            pltpu.VMEM((N_DEV, d, h_per), jnp.bfloat16),
            pltpu.VMEM((N_DEV, d, h_per), jnp.bfloat16),
            pltpu.VMEM((N_DEV, h_per, d), jnp.bfloat16),
            pltpu.VMEM((N_DEV, h_per, d), jnp.bfloat16),
            pltpu.VMEM((N_DEV, h_per, d), jnp.bfloat16),
            pltpu.SemaphoreType.DMA((N_RDMA,)),
            pltpu.SemaphoreType.DMA((N_RDMA,)),
        ],
        compiler_params=pltpu.CompilerParams(collective_id=0),
    )(x, Win0, Wout0, Win1, Wout1, Win2, Wout2)


# device time: 39439 ns/iter; 1.0944x vs baseline; 1.0004x over previous
import jax
import jax.numpy as jnp
from jax import lax
from jax.experimental import pallas as pl
from jax.experimental.pallas import tpu as pltpu

N_DEV = 4
N_RDMA = 18


def kernel(x, Win0, Wout0, Win1, Wout1, Win2, Wout2):
    m_per, d = x.shape
    _, h_per = Win0.shape

    def body(x_ref, win0, wout0, win1, wout1, win2, wout2, out_ref,
             W0, W1, W2, V0, V1, V2, send_sems, recv_sems):
        me = lax.axis_index("i")
        ypart = me ^ 1
        xpart = 3 - me

        sem = iter(range(N_RDMA))

        def start(*quads):
            rdmas = []
            for src, dst, b, partner in quads:
                i = next(sem)
                r = pltpu.make_async_remote_copy(
                    src_ref=src.at[b],
                    dst_ref=dst.at[b],
                    send_sem=send_sems.at[i],
                    recv_sem=recv_sems.at[i],
                    device_id=(partner,),
                    device_id_type=pl.DeviceIdType.MESH,
                )
                r.start()
                rdmas.append(r)
            return rdmas

        def wait(rdmas):
            for r in rdmas:
                r.wait()

        barrier = pltpu.get_barrier_semaphore()
        for nbr in (ypart, xpart):
            pl.semaphore_signal(
                barrier, inc=1,
                device_id=(nbr,), device_id_type=pl.DeviceIdType.MESH,
            )
        pl.semaphore_wait(barrier, 2)

        def stage(W, V, w_ref, v_ref):
            W[pl.ds(me, 1)] = w_ref[...].astype(jnp.bfloat16)[None]
            V[pl.ds(me, 1)] = v_ref[...].astype(jnp.bfloat16)[None]

        def own(W, V):
            return start(
                (W, W, me, ypart), (V, V, me, xpart),
                (W, W, me, xpart), (V, V, me, ypart),
            )

        def fwd(W, V):
            return start(
                (W, W, ypart, xpart), (V, V, xpart, ypart),
            )

        def chunk(W, V, q, xv):
            hq = jnp.maximum(
                jnp.dot(
                    xv, W[pl.ds(q, 1)][0],
                    preferred_element_type=jnp.float32,
                ),
                0.0,
            ).astype(jnp.bfloat16)
            return jnp.dot(
                hq, V[pl.ds(q, 1)][0], preferred_element_type=jnp.float32
            )

        def early(W, V, xv):
            return (
                chunk(W, V, me, xv)
                + chunk(W, V, ypart, xv)
                + chunk(W, V, xpart, xv)
            )

        stage(W0, V0, win0, wout0)
        r1_0 = own(W0, V0)
        stage(W1, V1, win1, wout1)
        r1_1 = own(W1, V1)
        stage(W2, V2, win2, wout2)
        xl = x_ref[...].astype(jnp.bfloat16)
        wait(r1_0)
        f0 = fwd(W0, V0)
        r1_2 = own(W2, V2)
        acc0 = early(W0, V0, xl)
        wait(r1_1)
        f1 = fwd(W1, V1)
        wait(f0)
        xl1 = (acc0 + chunk(W0, V0, me ^ 2, xl)).astype(jnp.bfloat16)
        acc1 = early(W1, V1, xl1)
        wait(r1_2)
        f2 = fwd(W2, V2)
        wait(f1)
        xl2 = (acc1 + chunk(W1, V1, me ^ 2, xl1)).astype(jnp.bfloat16)
        acc2 = early(W2, V2, xl2)
        wait(f2)
        out_ref[...] = acc2 + chunk(W2, V2, me ^ 2, xl2)

    return pl.pallas_call(
        body,
        out_shape=jax.ShapeDtypeStruct((m_per, d), jnp.float32),
        in_specs=[pl.BlockSpec(memory_space=pltpu.VMEM)] * 7,
        out_specs=pl.BlockSpec(memory_space=pltpu.VMEM),
        scratch_shapes=[
            pltpu.VMEM((N_DEV, d, h_per), jnp.bfloat16),
            pltpu.VMEM((N_DEV, d, h_per), jnp.bfloat16),
            pltpu.VMEM((N_DEV, d, h_per), jnp.bfloat16),
            pltpu.VMEM((N_DEV, h_per, d), jnp.bfloat16),
            pltpu.VMEM((N_DEV, h_per, d), jnp.bfloat16),
            pltpu.VMEM((N_DEV, h_per, d), jnp.bfloat16),
            pltpu.SemaphoreType.DMA((N_RDMA,)),
            pltpu.SemaphoreType.DMA((N_RDMA,)),
        ],
        compiler_params=pltpu.CompilerParams(collective_id=0),
    )(x, Win0, Wout0, Win1, Wout1, Win2, Wout2)
